# Initial kernel scaffold; baseline (speedup 1.0000x reference)
import jax
import jax.numpy as jnp
from jax.experimental import pallas as pl


def kernel(x, edge_index, W_gcn, b_gcn, W_enc, b_enc, W_mu, b_mu, W_std, b_std):
    raise NotImplementedError("write your pallas kernel here")


# speedup vs baseline: 17.6611x; 17.6611x over previous
import functools

import jax
import jax.numpy as jnp
from jax import lax
from jax.experimental import pallas as pl
from jax.experimental.pallas import tpu as pltpu
from jax.experimental.pallas import tpu_sc as plsc

N = 10000
E = 320000
D = 128
DEPTH = 5

NC = 2
NS = 16
NW = NC * NS
E_PER = E // NW
DEG_CH = 2000
EDGE_CH = 80
N_CHUNKS = E_PER // EDGE_CH
ROWS_PER_TILE = N // NS

@functools.cache
def _make_deg():
    mesh = plsc.VectorSubcoreMesh(core_axis_name="c", subcore_axis_name="s",
                                  num_cores=NC, num_subcores=NS)
    return pl.kernel(
        _deg_body,
        out_type=jax.ShapeDtypeStruct((NC, N), jnp.float32),
        mesh=mesh,
        scratch_types=[
            pltpu.VMEM((EDGE_CH,), jnp.int32),
            pltpu.VMEM((EDGE_CH,), jnp.float32),
            pltpu.VMEM_SHARED((N,), jnp.float32),
        ],
        compiler_params=pltpu.CompilerParams(use_tc_tiling_on_sc=False),
    )


def _deg_body(dst_hbm, zeros1_hbm, out_hbm, didx_v, ones_v, deg_s):
    cid = lax.axis_index("c")
    sid = lax.axis_index("s")
    wid = sid * NC + cid
    base = wid * E_PER

    def fill(i, _):
        ones_v[pl.ds(i * 16, 16)] = jnp.full((16,), 1.0, jnp.float32)
        return 0

    lax.fori_loop(0, EDGE_CH // 16, fill, 0)

    @pl.when(sid < 10)
    def _():
        pltpu.sync_copy(zeros1_hbm.at[pl.ds(sid * 1000, 1000)],
                        deg_s.at[pl.ds(sid * 1000, 1000)])

    plsc.subcore_barrier()

    def chunk(c, _):
        pltpu.sync_copy(dst_hbm.at[pl.ds(base + c * EDGE_CH, EDGE_CH)],
                        didx_v)
        pltpu.sync_copy(ones_v, deg_s.at[didx_v], add=True)
        return 0

    lax.fori_loop(0, N_CHUNKS, chunk, 0)
    plsc.subcore_barrier()

    @pl.when(sid < 10)
    def _():
        pltpu.sync_copy(deg_s.at[pl.ds(sid * 1000, 1000)],
                        out_hbm.at[cid, pl.ds(sid * 1000, 1000)])


@functools.cache
def _make_spmm():
    mesh = plsc.VectorSubcoreMesh(core_axis_name="c", subcore_axis_name="s",
                                  num_cores=NC, num_subcores=NS)
    return pl.kernel(
        _spmm_body,
        out_type=jax.ShapeDtypeStruct((NC, N, D), jnp.float32),
        mesh=mesh,
        scratch_types=[
            pltpu.VMEM((EDGE_CH,), jnp.int32),
            pltpu.VMEM((EDGE_CH,), jnp.int32),
            pltpu.VMEM((EDGE_CH, D), jnp.float32),
            pltpu.VMEM_SHARED((N, D), jnp.float32),
            pltpu.SemaphoreType.DMA,
        ],
        compiler_params=pltpu.CompilerParams(use_tc_tiling_on_sc=False),
    )


def _spmm_body(src_hbm, dst_hbm, g_hbm, zeros_hbm, out_hbm,
               sidx_v, didx_v, rows_v, acc_s, sem):
    cid = lax.axis_index("c")
    sid = lax.axis_index("s")
    wid = sid * NC + cid
    base = wid * E_PER
    row0 = sid * ROWS_PER_TILE

    pltpu.sync_copy(zeros_hbm.at[pl.ds(row0, ROWS_PER_TILE)],
                    acc_s.at[pl.ds(row0, ROWS_PER_TILE)])
    plsc.subcore_barrier()

    def chunk(c, _):
        e0 = base + c * EDGE_CH
        pltpu.sync_copy(src_hbm.at[pl.ds(e0, EDGE_CH)], sidx_v)
        pltpu.sync_copy(dst_hbm.at[pl.ds(e0, EDGE_CH)], didx_v)
        pltpu.async_copy(g_hbm.at[sidx_v], rows_v, sem).wait()
        pltpu.sync_copy(rows_v, acc_s.at[didx_v], add=True)
        return 0

    lax.fori_loop(0, N_CHUNKS, chunk, 0)
    plsc.subcore_barrier()
    pltpu.sync_copy(acc_s.at[pl.ds(row0, ROWS_PER_TILE)],
                    out_hbm.at[cid, pl.ds(row0, ROWS_PER_TILE)])


def _scale_body(x_ref, w_ref, degp_ref, g_ref, dinv_ref):
    h0 = jnp.dot(x_ref[...], w_ref[...], preferred_element_type=jnp.float32)
    deg = jnp.sum(degp_ref[...], axis=0) + 1.0
    dinv = lax.rsqrt(deg)
    g_ref[...] = h0 * dinv[:, None]
    dinv_ref[...] = dinv


def _mlp_body(accp_ref, g_ref, dinv_ref, bgcn_ref, wenc_ref, benc_ref,
              wmu_ref, bmu_ref, wstd_ref, bstd_ref, eps_ref,
              xn_ref, mu_ref, std_ref):
    acc = accp_ref[0] + accp_ref[1]
    h = dinv_ref[...][:, None] * (acc + g_ref[...]) + bgcn_ref[...][None, :]
    for i in range(DEPTH):
        z = jnp.dot(h, wenc_ref[i], preferred_element_type=jnp.float32)
        h = jax.nn.sigmoid(z + benc_ref[i][None, :])
    mu = jnp.dot(h, wmu_ref[...], preferred_element_type=jnp.float32)
    mu = mu + bmu_ref[...][None, :]
    s = jnp.dot(h, wstd_ref[...], preferred_element_type=jnp.float32)
    s = s + bstd_ref[...][None, :] - 5.0
    std = jnp.maximum(s, 0.0) + jnp.log1p(jnp.exp(-jnp.abs(s)))
    mu_ref[...] = mu
    std_ref[...] = std
    xn_ref[...] = mu + std * eps_ref[...]


_R = 1024
_G = (N + _R - 1) // _R


def kernel(x, edge_index, W_gcn, b_gcn, W_enc, b_enc, W_mu, b_mu, W_std,
           b_std):
    src = edge_index[0]
    dst = edge_index[1]

    zeros1 = jnp.zeros((N,), jnp.float32)
    deg_parts = _make_deg()(dst, zeros1)

    g, dinv = pl.pallas_call(
        _scale_body,
        grid=(_G,),
        in_specs=[
            pl.BlockSpec((_R, D), lambda i: (i, 0)),
            pl.BlockSpec((D, D), lambda i: (0, 0)),
            pl.BlockSpec((NC, _R), lambda i: (0, i)),
        ],
        out_specs=[
            pl.BlockSpec((_R, D), lambda i: (i, 0)),
            pl.BlockSpec((_R,), lambda i: (i,)),
        ],
        out_shape=[
            jax.ShapeDtypeStruct((N, D), jnp.float32),
            jax.ShapeDtypeStruct((N,), jnp.float32),
        ],
    )(x, W_gcn, deg_parts)

    zeros = jnp.zeros((N, D), jnp.float32)
    acc_parts = _make_spmm()(src, dst, g, zeros)

    eps = jax.random.uniform(jax.random.key(42), (N, D), dtype=jnp.float32)

    x_new, mu, std = pl.pallas_call(
        _mlp_body,
        grid=(_G,),
        in_specs=[
            pl.BlockSpec((NC, _R, D), lambda i: (0, i, 0)),
            pl.BlockSpec((_R, D), lambda i: (i, 0)),
            pl.BlockSpec((_R,), lambda i: (i,)),
            pl.BlockSpec((D,), lambda i: (0,)),
            pl.BlockSpec((DEPTH, D, D), lambda i: (0, 0, 0)),
            pl.BlockSpec((DEPTH, D), lambda i: (0, 0)),
            pl.BlockSpec((D, D), lambda i: (0, 0)),
            pl.BlockSpec((D,), lambda i: (0,)),
            pl.BlockSpec((D, D), lambda i: (0, 0)),
            pl.BlockSpec((D,), lambda i: (0,)),
            pl.BlockSpec((_R, D), lambda i: (i, 0)),
        ],
        out_specs=[
            pl.BlockSpec((_R, D), lambda i: (i, 0)),
            pl.BlockSpec((_R, D), lambda i: (i, 0)),
            pl.BlockSpec((_R, D), lambda i: (i, 0)),
        ],
        out_shape=[
            jax.ShapeDtypeStruct((N, D), jnp.float32),
            jax.ShapeDtypeStruct((N, D), jnp.float32),
            jax.ShapeDtypeStruct((N, D), jnp.float32),
        ],
    )(acc_parts, g, dinv, b_gcn, W_enc, b_enc, W_mu, b_mu, W_std, b_std, eps)

    return (x_new, mu, std)


# speedup vs baseline: 31.6947x; 1.7946x over previous
import functools

import jax
import jax.numpy as jnp
from jax import lax
from jax.experimental import pallas as pl
from jax.experimental.pallas import tpu as pltpu
from jax.experimental.pallas import tpu_sc as plsc

N = 10000
E = 320000
D = 128
DEPTH = 5

NC = 2
NS = 16
NW = NC * NS
E_PER = E // NW
DEG_CH = 2000
EDGE_CH = 80
N_CHUNKS = E_PER // EDGE_CH
GK = 4
N_GROUPS = N_CHUNKS // GK
N_REM = N_CHUNKS - N_GROUPS * GK
ROWS_PER_TILE = N // NS

@functools.cache
def _make_deg():
    mesh = plsc.VectorSubcoreMesh(core_axis_name="c", subcore_axis_name="s",
                                  num_cores=NC, num_subcores=NS)
    return pl.kernel(
        _deg_body,
        out_type=jax.ShapeDtypeStruct((NC, N), jnp.float32),
        mesh=mesh,
        scratch_types=[
            pltpu.VMEM((E_PER,), jnp.int32),
            pltpu.VMEM((EDGE_CH,), jnp.float32),
            pltpu.VMEM_SHARED((N,), jnp.float32),
            pltpu.SemaphoreType.DMA,
        ],
        compiler_params=pltpu.CompilerParams(use_tc_tiling_on_sc=False),
    )


def _deg_body(dst_hbm, zeros1_hbm, out_hbm, didx_v, ones_v, deg_s, sem):
    cid = lax.axis_index("c")
    sid = lax.axis_index("s")
    wid = sid * NC + cid
    base = wid * E_PER

    def fill(i, _):
        ones_v[pl.ds(i * 16, 16)] = jnp.full((16,), 1.0, jnp.float32)
        return 0

    lax.fori_loop(0, EDGE_CH // 16, fill, 0)

    @pl.when(sid < 10)
    def _():
        pltpu.sync_copy(zeros1_hbm.at[pl.ds(sid * 1000, 1000)],
                        deg_s.at[pl.ds(sid * 1000, 1000)])

    pltpu.sync_copy(dst_hbm.at[pl.ds(base, E_PER)], didx_v)
    plsc.subcore_barrier()

    def chunk(c, _):
        pltpu.async_copy(
            ones_v, deg_s.at[didx_v.at[pl.ds(c * EDGE_CH, EDGE_CH)]], sem,
            add=True)
        return 0

    lax.fori_loop(0, N_CHUNKS, chunk, 0)
    pltpu.make_async_copy(dst_hbm.at[pl.ds(base, E_PER)], didx_v, sem).wait()
    plsc.subcore_barrier()

    @pl.when(sid < 10)
    def _():
        pltpu.sync_copy(deg_s.at[pl.ds(sid * 1000, 1000)],
                        out_hbm.at[cid, pl.ds(sid * 1000, 1000)])


@functools.cache
def _make_spmm():
    mesh = plsc.VectorSubcoreMesh(core_axis_name="c", subcore_axis_name="s",
                                  num_cores=NC, num_subcores=NS)
    return pl.kernel(
        _spmm_body,
        out_type=jax.ShapeDtypeStruct((NC, N, D), jnp.float32),
        mesh=mesh,
        scratch_types=[
            pltpu.VMEM((GK * EDGE_CH,), jnp.int32),
            pltpu.VMEM((GK * EDGE_CH,), jnp.int32),
            pltpu.VMEM((GK, EDGE_CH, D), jnp.float32),
            pltpu.VMEM_SHARED((N, D), jnp.float32),
            pltpu.SemaphoreType.DMA,
            pltpu.SemaphoreType.DMA,
        ],
        compiler_params=pltpu.CompilerParams(use_tc_tiling_on_sc=False),
    )


def _spmm_body(src_hbm, dst_hbm, g_hbm, zeros_hbm, out_hbm,
               sidx_v, didx_v, rows_v, acc_s, semg, sems):
    cid = lax.axis_index("c")
    sid = lax.axis_index("s")
    wid = sid * NC + cid
    base = wid * E_PER
    row0 = sid * ROWS_PER_TILE

    pltpu.sync_copy(zeros_hbm.at[pl.ds(row0, ROWS_PER_TILE)],
                    acc_s.at[pl.ds(row0, ROWS_PER_TILE)])
    plsc.subcore_barrier()

    def do_group(e0, nk):
        pltpu.sync_copy(src_hbm.at[pl.ds(base + e0, nk * EDGE_CH)],
                        sidx_v.at[pl.ds(0, nk * EDGE_CH)])
        pltpu.sync_copy(dst_hbm.at[pl.ds(base + e0, nk * EDGE_CH)],
                        didx_v.at[pl.ds(0, nk * EDGE_CH)])
        gathers = []
        for j in range(nk):
            gathers.append(pltpu.async_copy(
                g_hbm.at[sidx_v.at[pl.ds(j * EDGE_CH, EDGE_CH)]],
                rows_v.at[j], semg))
        for g in gathers:
            g.wait()
        scatters = []
        for j in range(nk):
            scatters.append(pltpu.async_copy(
                rows_v.at[j],
                acc_s.at[didx_v.at[pl.ds(j * EDGE_CH, EDGE_CH)]],
                sems, add=True))
        for s in scatters:
            s.wait()

    def group(gi, _):
        do_group(gi * (GK * EDGE_CH), GK)
        return 0

    lax.fori_loop(0, N_GROUPS, group, 0)
    if N_REM:
        do_group(N_GROUPS * GK * EDGE_CH, N_REM)
    plsc.subcore_barrier()
    pltpu.sync_copy(acc_s.at[pl.ds(row0, ROWS_PER_TILE)],
                    out_hbm.at[cid, pl.ds(row0, ROWS_PER_TILE)])


def _scale_body(x_ref, w_ref, degp_ref, g_ref, dinv_ref):
    h0 = jnp.dot(x_ref[...], w_ref[...], preferred_element_type=jnp.float32)
    deg = jnp.sum(degp_ref[...], axis=0) + 1.0
    dinv = lax.rsqrt(deg)
    g_ref[...] = h0 * dinv[:, None]
    dinv_ref[...] = dinv


def _mlp_body(accp_ref, g_ref, dinv_ref, bgcn_ref, wenc_ref, benc_ref,
              wmu_ref, bmu_ref, wstd_ref, bstd_ref, eps_ref,
              xn_ref, mu_ref, std_ref):
    acc = accp_ref[0] + accp_ref[1]
    h = dinv_ref[...][:, None] * (acc + g_ref[...]) + bgcn_ref[...][None, :]
    for i in range(DEPTH):
        z = jnp.dot(h, wenc_ref[i], preferred_element_type=jnp.float32)
        h = jax.nn.sigmoid(z + benc_ref[i][None, :])
    mu = jnp.dot(h, wmu_ref[...], preferred_element_type=jnp.float32)
    mu = mu + bmu_ref[...][None, :]
    s = jnp.dot(h, wstd_ref[...], preferred_element_type=jnp.float32)
    s = s + bstd_ref[...][None, :] - 5.0
    std = jnp.maximum(s, 0.0) + jnp.log1p(jnp.exp(-jnp.abs(s)))
    mu_ref[...] = mu
    std_ref[...] = std
    xn_ref[...] = mu + std * eps_ref[...]


_R = 1024
_G = (N + _R - 1) // _R


def kernel(x, edge_index, W_gcn, b_gcn, W_enc, b_enc, W_mu, b_mu, W_std,
           b_std):
    src = edge_index[0]
    dst = edge_index[1]

    zeros1 = jnp.zeros((N,), jnp.float32)
    deg_parts = _make_deg()(dst, zeros1)

    g, dinv = pl.pallas_call(
        _scale_body,
        grid=(_G,),
        in_specs=[
            pl.BlockSpec((_R, D), lambda i: (i, 0)),
            pl.BlockSpec((D, D), lambda i: (0, 0)),
            pl.BlockSpec((NC, _R), lambda i: (0, i)),
        ],
        out_specs=[
            pl.BlockSpec((_R, D), lambda i: (i, 0)),
            pl.BlockSpec((_R,), lambda i: (i,)),
        ],
        out_shape=[
            jax.ShapeDtypeStruct((N, D), jnp.float32),
            jax.ShapeDtypeStruct((N,), jnp.float32),
        ],
    )(x, W_gcn, deg_parts)

    zeros = jnp.zeros((N, D), jnp.float32)
    acc_parts = _make_spmm()(src, dst, g, zeros)

    eps = jax.random.uniform(jax.random.key(42), (N, D), dtype=jnp.float32)

    x_new, mu, std = pl.pallas_call(
        _mlp_body,
        grid=(_G,),
        in_specs=[
            pl.BlockSpec((NC, _R, D), lambda i: (0, i, 0)),
            pl.BlockSpec((_R, D), lambda i: (i, 0)),
            pl.BlockSpec((_R,), lambda i: (i,)),
            pl.BlockSpec((D,), lambda i: (0,)),
            pl.BlockSpec((DEPTH, D, D), lambda i: (0, 0, 0)),
            pl.BlockSpec((DEPTH, D), lambda i: (0, 0)),
            pl.BlockSpec((D, D), lambda i: (0, 0)),
            pl.BlockSpec((D,), lambda i: (0,)),
            pl.BlockSpec((D, D), lambda i: (0, 0)),
            pl.BlockSpec((D,), lambda i: (0,)),
            pl.BlockSpec((_R, D), lambda i: (i, 0)),
        ],
        out_specs=[
            pl.BlockSpec((_R, D), lambda i: (i, 0)),
            pl.BlockSpec((_R, D), lambda i: (i, 0)),
            pl.BlockSpec((_R, D), lambda i: (i, 0)),
        ],
        out_shape=[
            jax.ShapeDtypeStruct((N, D), jnp.float32),
            jax.ShapeDtypeStruct((N, D), jnp.float32),
            jax.ShapeDtypeStruct((N, D), jnp.float32),
        ],
    )(acc_parts, g, dinv, b_gcn, W_enc, b_enc, W_mu, b_mu, W_std, b_std, eps)

    return (x_new, mu, std)


# speedup vs baseline: 38.9327x; 1.2284x over previous
import functools

import jax
import jax.numpy as jnp
from jax import lax
from jax.experimental import pallas as pl
from jax.experimental.pallas import tpu as pltpu
from jax.experimental.pallas import tpu_sc as plsc

N = 10000
E = 320000
D = 128
DEPTH = 5

NC = 2
NS = 16
NW = NC * NS
E_PER = E // NW
DEG_CH = 2000
EDGE_CH = 80
N_CHUNKS = E_PER // EDGE_CH
SP_CH = 96
GK = 2
GE = GK * SP_CH
N_PAIRS = E_PER // (2 * GE)
SP_REM = E_PER - N_PAIRS * 2 * GE
ROWS_PER_TILE = N // NS

@functools.cache
def _make_deg():
    mesh = plsc.VectorSubcoreMesh(core_axis_name="c", subcore_axis_name="s",
                                  num_cores=NC, num_subcores=NS)
    return pl.kernel(
        _deg_body,
        out_type=jax.ShapeDtypeStruct((NC, N), jnp.float32),
        mesh=mesh,
        scratch_types=[
            pltpu.VMEM((E_PER,), jnp.int32),
            pltpu.VMEM((EDGE_CH,), jnp.float32),
            pltpu.VMEM_SHARED((N,), jnp.float32),
            pltpu.SemaphoreType.DMA,
        ],
        compiler_params=pltpu.CompilerParams(use_tc_tiling_on_sc=False),
    )


def _deg_body(dst_hbm, zeros1_hbm, out_hbm, didx_v, ones_v, deg_s, sem):
    cid = lax.axis_index("c")
    sid = lax.axis_index("s")
    wid = sid * NC + cid
    base = wid * E_PER

    def fill(i, _):
        ones_v[pl.ds(i * 16, 16)] = jnp.full((16,), 1.0, jnp.float32)
        return 0

    lax.fori_loop(0, EDGE_CH // 16, fill, 0)

    @pl.when(sid < 10)
    def _():
        pltpu.sync_copy(zeros1_hbm.at[pl.ds(sid * 1000, 1000)],
                        deg_s.at[pl.ds(sid * 1000, 1000)])

    pltpu.sync_copy(dst_hbm.at[pl.ds(base, E_PER)], didx_v)
    plsc.subcore_barrier()

    def chunk(c, _):
        pltpu.async_copy(
            ones_v, deg_s.at[didx_v.at[pl.ds(c * EDGE_CH, EDGE_CH)]], sem,
            add=True)
        return 0

    lax.fori_loop(0, N_CHUNKS, chunk, 0)
    pltpu.make_async_copy(dst_hbm.at[pl.ds(base, E_PER)], didx_v, sem).wait()
    plsc.subcore_barrier()

    @pl.when(sid < 10)
    def _():
        pltpu.sync_copy(deg_s.at[pl.ds(sid * 1000, 1000)],
                        out_hbm.at[cid, pl.ds(sid * 1000, 1000)])


@functools.cache
def _make_spmm():
    mesh = plsc.VectorSubcoreMesh(core_axis_name="c", subcore_axis_name="s",
                                  num_cores=NC, num_subcores=NS)
    return pl.kernel(
        _spmm_body,
        out_type=jax.ShapeDtypeStruct((NC, N, D), jnp.float32),
        mesh=mesh,
        scratch_types=[
            pltpu.VMEM((GE,), jnp.int32),
            pltpu.VMEM((GE,), jnp.int32),
            pltpu.VMEM((GE,), jnp.int32),
            pltpu.VMEM((GE,), jnp.int32),
            pltpu.VMEM((GK, SP_CH, D), jnp.float32),
            pltpu.VMEM((GK, SP_CH, D), jnp.float32),
            pltpu.VMEM_SHARED((N, D), jnp.float32),
            pltpu.SemaphoreType.DMA,
            pltpu.SemaphoreType.DMA,
            pltpu.SemaphoreType.DMA,
            pltpu.SemaphoreType.DMA,
        ],
        compiler_params=pltpu.CompilerParams(use_tc_tiling_on_sc=False),
    )


def _spmm_body(src_hbm, dst_hbm, g_hbm, zeros_hbm, out_hbm,
               sidx0, sidx1, didx0, didx1, rows0, rows1, acc_s,
               semg0, semg1, sems0, sems1):
    cid = lax.axis_index("c")
    sid = lax.axis_index("s")
    wid = sid * NC + cid
    base = wid * E_PER
    row0 = sid * ROWS_PER_TILE

    sidx = (sidx0, sidx1)
    didx = (didx0, didx1)
    rows = (rows0, rows1)
    semg = (semg0, semg1)
    sems = (sems0, sems1)

    pltpu.sync_copy(zeros_hbm.at[pl.ds(row0, ROWS_PER_TILE)],
                    acc_s.at[pl.ds(row0, ROWS_PER_TILE)])
    plsc.subcore_barrier()

    def stage_idx(s, e0):
        pltpu.sync_copy(src_hbm.at[pl.ds(base + e0, GE)], sidx[s])
        pltpu.sync_copy(dst_hbm.at[pl.ds(base + e0, GE)], didx[s])

    def fire_gathers(s):
        for j in range(GK):
            pltpu.async_copy(
                g_hbm.at[sidx[s].at[pl.ds(j * SP_CH, SP_CH)]],
                rows[s].at[j], semg[s])

    def wait_gathers(s):
        for j in range(GK):
            pltpu.make_async_copy(zeros_hbm.at[pl.ds(0, SP_CH)],
                                  rows[s].at[j], semg[s]).wait()

    def fire_scatters(s):
        for j in range(GK):
            pltpu.async_copy(
                rows[s].at[j],
                acc_s.at[didx[s].at[pl.ds(j * SP_CH, SP_CH)]],
                sems[s], add=True)

    def drain_scatters(s):
        for j in range(GK):
            pltpu.make_async_copy(zeros_hbm.at[pl.ds(0, SP_CH)],
                                  rows[s].at[j], sems[s]).wait()

    def pair(i, _):
        g0 = i * 2 * GE

        @pl.when(i >= 1)
        def _():
            drain_scatters(0)

        stage_idx(0, g0)
        fire_gathers(0)

        @pl.when(i >= 1)
        def _():
            wait_gathers(1)
            fire_scatters(1)
            drain_scatters(1)

        stage_idx(1, g0 + GE)
        fire_gathers(1)
        wait_gathers(0)
        fire_scatters(0)
        return 0

    lax.fori_loop(0, N_PAIRS, pair, 0)

    drain_scatters(0)
    wait_gathers(1)
    fire_scatters(1)
    if SP_REM:
        eL = base + N_PAIRS * 2 * GE
        pltpu.sync_copy(src_hbm.at[pl.ds(eL, SP_REM)],
                        sidx0.at[pl.ds(0, SP_REM)])
        pltpu.sync_copy(dst_hbm.at[pl.ds(eL, SP_REM)],
                        didx0.at[pl.ds(0, SP_REM)])
        pltpu.async_copy(g_hbm.at[sidx0.at[pl.ds(0, SP_REM)]],
                         rows0.at[0, pl.ds(0, SP_REM)], semg0).wait()
        pltpu.sync_copy(rows0.at[0, pl.ds(0, SP_REM)],
                        acc_s.at[didx0.at[pl.ds(0, SP_REM)]], add=True)
    drain_scatters(1)
    plsc.subcore_barrier()
    pltpu.sync_copy(acc_s.at[pl.ds(row0, ROWS_PER_TILE)],
                    out_hbm.at[cid, pl.ds(row0, ROWS_PER_TILE)])


def _scale_body(x_ref, w_ref, degp_ref, g_ref, dinv_ref):
    h0 = jnp.dot(x_ref[...], w_ref[...], preferred_element_type=jnp.float32)
    deg = jnp.sum(degp_ref[...], axis=0) + 1.0
    dinv = lax.rsqrt(deg)
    g_ref[...] = h0 * dinv[:, None]
    dinv_ref[...] = dinv


def _mlp_body(accp_ref, g_ref, dinv_ref, bgcn_ref, wenc_ref, benc_ref,
              wmu_ref, bmu_ref, wstd_ref, bstd_ref, eps_ref,
              xn_ref, mu_ref, std_ref):
    acc = accp_ref[0] + accp_ref[1]
    h = dinv_ref[...][:, None] * (acc + g_ref[...]) + bgcn_ref[...][None, :]
    for i in range(DEPTH):
        z = jnp.dot(h, wenc_ref[i], preferred_element_type=jnp.float32)
        h = jax.nn.sigmoid(z + benc_ref[i][None, :])
    mu = jnp.dot(h, wmu_ref[...], preferred_element_type=jnp.float32)
    mu = mu + bmu_ref[...][None, :]
    s = jnp.dot(h, wstd_ref[...], preferred_element_type=jnp.float32)
    s = s + bstd_ref[...][None, :] - 5.0
    std = jnp.maximum(s, 0.0) + jnp.log1p(jnp.exp(-jnp.abs(s)))
    mu_ref[...] = mu
    std_ref[...] = std
    xn_ref[...] = mu + std * eps_ref[...]


_R = 1024
_G = (N + _R - 1) // _R


def kernel(x, edge_index, W_gcn, b_gcn, W_enc, b_enc, W_mu, b_mu, W_std,
           b_std):
    src = edge_index[0]
    dst = edge_index[1]

    zeros1 = jnp.zeros((N,), jnp.float32)
    deg_parts = _make_deg()(dst, zeros1)

    g, dinv = pl.pallas_call(
        _scale_body,
        grid=(_G,),
        in_specs=[
            pl.BlockSpec((_R, D), lambda i: (i, 0)),
            pl.BlockSpec((D, D), lambda i: (0, 0)),
            pl.BlockSpec((NC, _R), lambda i: (0, i)),
        ],
        out_specs=[
            pl.BlockSpec((_R, D), lambda i: (i, 0)),
            pl.BlockSpec((_R,), lambda i: (i,)),
        ],
        out_shape=[
            jax.ShapeDtypeStruct((N, D), jnp.float32),
            jax.ShapeDtypeStruct((N,), jnp.float32),
        ],
    )(x, W_gcn, deg_parts)

    zeros = jnp.zeros((N, D), jnp.float32)
    acc_parts = _make_spmm()(src, dst, g, zeros)

    eps = jax.random.uniform(jax.random.key(42), (N, D), dtype=jnp.float32)

    x_new, mu, std = pl.pallas_call(
        _mlp_body,
        grid=(_G,),
        in_specs=[
            pl.BlockSpec((NC, _R, D), lambda i: (0, i, 0)),
            pl.BlockSpec((_R, D), lambda i: (i, 0)),
            pl.BlockSpec((_R,), lambda i: (i,)),
            pl.BlockSpec((D,), lambda i: (0,)),
            pl.BlockSpec((DEPTH, D, D), lambda i: (0, 0, 0)),
            pl.BlockSpec((DEPTH, D), lambda i: (0, 0)),
            pl.BlockSpec((D, D), lambda i: (0, 0)),
            pl.BlockSpec((D,), lambda i: (0,)),
            pl.BlockSpec((D, D), lambda i: (0, 0)),
            pl.BlockSpec((D,), lambda i: (0,)),
            pl.BlockSpec((_R, D), lambda i: (i, 0)),
        ],
        out_specs=[
            pl.BlockSpec((_R, D), lambda i: (i, 0)),
            pl.BlockSpec((_R, D), lambda i: (i, 0)),
            pl.BlockSpec((_R, D), lambda i: (i, 0)),
        ],
        out_shape=[
            jax.ShapeDtypeStruct((N, D), jnp.float32),
            jax.ShapeDtypeStruct((N, D), jnp.float32),
            jax.ShapeDtypeStruct((N, D), jnp.float32),
        ],
    )(acc_parts, g, dinv, b_gcn, W_enc, b_enc, W_mu, b_mu, W_std, b_std, eps)

    return (x_new, mu, std)


# speedup vs baseline: 39.1367x; 1.0052x over previous
import functools

import jax
import jax.numpy as jnp
import numpy as np
from jax import lax
from jax.experimental import pallas as pl
from jax.experimental.pallas import tpu as pltpu
from jax.experimental.pallas import tpu_sc as plsc

N = 10000
E = 320000
D = 128
DEPTH = 5

NC = 2
NS = 16
NW = NC * NS
E_PER = E // NW
DEG_CH = 2000
EDGE_CH = 80
N_CHUNKS = E_PER // EDGE_CH
SP_CH = 96
GK = 2
GE = GK * SP_CH
N_PAIRS = E_PER // (2 * GE)
SP_REM = E_PER - N_PAIRS * 2 * GE
ROWS_PER_TILE = N // NS

@functools.cache
def _make_deg():
    mesh = plsc.VectorSubcoreMesh(core_axis_name="c", subcore_axis_name="s",
                                  num_cores=NC, num_subcores=NS)
    return pl.kernel(
        _deg_body,
        out_type=jax.ShapeDtypeStruct((NC, N), jnp.float32),
        mesh=mesh,
        scratch_types=[
            pltpu.VMEM((E_PER,), jnp.int32),
            pltpu.VMEM((EDGE_CH,), jnp.float32),
            pltpu.VMEM_SHARED((N,), jnp.float32),
            pltpu.SemaphoreType.DMA,
        ],
        compiler_params=pltpu.CompilerParams(use_tc_tiling_on_sc=False),
    )


def _deg_body(dst_hbm, zeros1_hbm, out_hbm, didx_v, ones_v, deg_s, sem):
    cid = lax.axis_index("c")
    sid = lax.axis_index("s")
    wid = sid * NC + cid
    base = wid * E_PER

    def fill(i, _):
        ones_v[pl.ds(i * 16, 16)] = jnp.full((16,), 1.0, jnp.float32)
        return 0

    lax.fori_loop(0, EDGE_CH // 16, fill, 0)

    @pl.when(sid < 10)
    def _():
        pltpu.sync_copy(zeros1_hbm.at[pl.ds(sid * 1000, 1000)],
                        deg_s.at[pl.ds(sid * 1000, 1000)])

    pltpu.sync_copy(dst_hbm.at[pl.ds(base, E_PER)], didx_v)
    plsc.subcore_barrier()

    def chunk(c, _):
        pltpu.async_copy(
            ones_v, deg_s.at[didx_v.at[pl.ds(c * EDGE_CH, EDGE_CH)]], sem,
            add=True)
        return 0

    lax.fori_loop(0, N_CHUNKS, chunk, 0)
    pltpu.make_async_copy(dst_hbm.at[pl.ds(base, E_PER)], didx_v, sem).wait()
    plsc.subcore_barrier()

    @pl.when(sid < 10)
    def _():
        pltpu.sync_copy(deg_s.at[pl.ds(sid * 1000, 1000)],
                        out_hbm.at[cid, pl.ds(sid * 1000, 1000)])


@functools.cache
def _make_spmm():
    mesh = plsc.VectorSubcoreMesh(core_axis_name="c", subcore_axis_name="s",
                                  num_cores=NC, num_subcores=NS)
    return pl.kernel(
        _spmm_body,
        out_type=jax.ShapeDtypeStruct((NC, N, D), jnp.float32),
        mesh=mesh,
        scratch_types=[
            pltpu.VMEM((GE,), jnp.int32),
            pltpu.VMEM((GE,), jnp.int32),
            pltpu.VMEM((GE,), jnp.int32),
            pltpu.VMEM((GE,), jnp.int32),
            pltpu.VMEM((GK, SP_CH, D), jnp.float32),
            pltpu.VMEM((GK, SP_CH, D), jnp.float32),
            pltpu.VMEM_SHARED((N, D), jnp.float32),
            pltpu.SemaphoreType.DMA,
            pltpu.SemaphoreType.DMA,
            pltpu.SemaphoreType.DMA,
            pltpu.SemaphoreType.DMA,
        ],
        compiler_params=pltpu.CompilerParams(use_tc_tiling_on_sc=False),
    )


def _spmm_body(src_hbm, dst_hbm, g_hbm, zeros_hbm, out_hbm,
               sidx0, sidx1, didx0, didx1, rows0, rows1, acc_s,
               semg0, semg1, sems0, sems1):
    cid = lax.axis_index("c")
    sid = lax.axis_index("s")
    wid = sid * NC + cid
    base = wid * E_PER
    row0 = sid * ROWS_PER_TILE

    sidx = (sidx0, sidx1)
    didx = (didx0, didx1)
    rows = (rows0, rows1)
    semg = (semg0, semg1)
    sems = (sems0, sems1)

    pltpu.sync_copy(zeros_hbm.at[pl.ds(row0, ROWS_PER_TILE)],
                    acc_s.at[pl.ds(row0, ROWS_PER_TILE)])
    plsc.subcore_barrier()

    def stage_idx(s, e0):
        pltpu.sync_copy(src_hbm.at[pl.ds(base + e0, GE)], sidx[s])
        pltpu.sync_copy(dst_hbm.at[pl.ds(base + e0, GE)], didx[s])

    def fire_gathers(s):
        for j in range(GK):
            pltpu.async_copy(
                g_hbm.at[sidx[s].at[pl.ds(j * SP_CH, SP_CH)]],
                rows[s].at[j], semg[s])

    def wait_gathers(s):
        for j in range(GK):
            pltpu.make_async_copy(zeros_hbm.at[pl.ds(0, SP_CH)],
                                  rows[s].at[j], semg[s]).wait()

    def fire_scatters(s):
        for j in range(GK):
            pltpu.async_copy(
                rows[s].at[j],
                acc_s.at[didx[s].at[pl.ds(j * SP_CH, SP_CH)]],
                sems[s], add=True)

    def drain_scatters(s):
        for j in range(GK):
            pltpu.make_async_copy(zeros_hbm.at[pl.ds(0, SP_CH)],
                                  rows[s].at[j], sems[s]).wait()

    def pair(i, _):
        g0 = i * 2 * GE

        @pl.when(i >= 1)
        def _():
            drain_scatters(0)

        stage_idx(0, g0)
        fire_gathers(0)

        @pl.when(i >= 1)
        def _():
            wait_gathers(1)
            fire_scatters(1)
            drain_scatters(1)

        stage_idx(1, g0 + GE)
        fire_gathers(1)
        wait_gathers(0)
        fire_scatters(0)
        return 0

    lax.fori_loop(0, N_PAIRS, pair, 0)

    drain_scatters(0)
    wait_gathers(1)
    fire_scatters(1)
    if SP_REM:
        eL = base + N_PAIRS * 2 * GE
        pltpu.sync_copy(src_hbm.at[pl.ds(eL, SP_REM)],
                        sidx0.at[pl.ds(0, SP_REM)])
        pltpu.sync_copy(dst_hbm.at[pl.ds(eL, SP_REM)],
                        didx0.at[pl.ds(0, SP_REM)])
        pltpu.async_copy(g_hbm.at[sidx0.at[pl.ds(0, SP_REM)]],
                         rows0.at[0, pl.ds(0, SP_REM)], semg0).wait()
        pltpu.sync_copy(rows0.at[0, pl.ds(0, SP_REM)],
                        acc_s.at[didx0.at[pl.ds(0, SP_REM)]], add=True)
    drain_scatters(1)
    plsc.subcore_barrier()
    pltpu.sync_copy(acc_s.at[pl.ds(row0, ROWS_PER_TILE)],
                    out_hbm.at[cid, pl.ds(row0, ROWS_PER_TILE)])


def _scale_body(x_ref, w_ref, degp_ref, g_ref, dinv_ref):
    h0 = jnp.dot(x_ref[...], w_ref[...], preferred_element_type=jnp.float32)
    deg = jnp.sum(degp_ref[...], axis=0) + 1.0
    dinv = lax.rsqrt(deg)
    g_ref[...] = h0 * dinv[:, None]
    dinv_ref[...] = dinv


def _mlp_body(accp_ref, g_ref, dinv_ref, bgcn_ref, wenc_ref, benc_ref,
              wmu_ref, bmu_ref, wstd_ref, bstd_ref, eps_ref,
              xn_ref, mu_ref, std_ref):
    acc = accp_ref[0] + accp_ref[1]
    h = dinv_ref[...][:, None] * (acc + g_ref[...]) + bgcn_ref[...][None, :]
    for i in range(DEPTH):
        z = jnp.dot(h, wenc_ref[i], preferred_element_type=jnp.float32)
        h = jax.nn.sigmoid(z + benc_ref[i][None, :])
    mu = jnp.dot(h, wmu_ref[...], preferred_element_type=jnp.float32)
    mu = mu + bmu_ref[...][None, :]
    s = jnp.dot(h, wstd_ref[...], preferred_element_type=jnp.float32)
    s = s + bstd_ref[...][None, :] - 5.0
    std = jnp.maximum(s, 0.0) + jnp.log1p(jnp.exp(-jnp.abs(s)))
    mu_ref[...] = mu
    std_ref[...] = std
    xn_ref[...] = mu + std * eps_ref[...]


_R = 1024
_G = (N + _R - 1) // _R


@functools.cache
def _eps_np():
    with jax.ensure_compile_time_eval():
        with jax.default_device(jax.devices("cpu")[0]):
            return np.asarray(jax.random.uniform(
                jax.random.key(42), (N, D), dtype=jnp.float32))


_ZEROS_ND = np.zeros((N, D), np.float32)
_ZEROS_N = np.zeros((N,), np.float32)


def kernel(x, edge_index, W_gcn, b_gcn, W_enc, b_enc, W_mu, b_mu, W_std,
           b_std):
    src = edge_index[0]
    dst = edge_index[1]

    deg_parts = _make_deg()(dst, _ZEROS_N)

    g, dinv = pl.pallas_call(
        _scale_body,
        grid=(_G,),
        in_specs=[
            pl.BlockSpec((_R, D), lambda i: (i, 0)),
            pl.BlockSpec((D, D), lambda i: (0, 0)),
            pl.BlockSpec((NC, _R), lambda i: (0, i)),
        ],
        out_specs=[
            pl.BlockSpec((_R, D), lambda i: (i, 0)),
            pl.BlockSpec((_R,), lambda i: (i,)),
        ],
        out_shape=[
            jax.ShapeDtypeStruct((N, D), jnp.float32),
            jax.ShapeDtypeStruct((N,), jnp.float32),
        ],
    )(x, W_gcn, deg_parts)

    acc_parts = _make_spmm()(src, dst, g, _ZEROS_ND)

    eps = _eps_np()

    x_new, mu, std = pl.pallas_call(
        _mlp_body,
        grid=(_G,),
        in_specs=[
            pl.BlockSpec((NC, _R, D), lambda i: (0, i, 0)),
            pl.BlockSpec((_R, D), lambda i: (i, 0)),
            pl.BlockSpec((_R,), lambda i: (i,)),
            pl.BlockSpec((D,), lambda i: (0,)),
            pl.BlockSpec((DEPTH, D, D), lambda i: (0, 0, 0)),
            pl.BlockSpec((DEPTH, D), lambda i: (0, 0)),
            pl.BlockSpec((D, D), lambda i: (0, 0)),
            pl.BlockSpec((D,), lambda i: (0,)),
            pl.BlockSpec((D, D), lambda i: (0, 0)),
            pl.BlockSpec((D,), lambda i: (0,)),
            pl.BlockSpec((_R, D), lambda i: (i, 0)),
        ],
        out_specs=[
            pl.BlockSpec((_R, D), lambda i: (i, 0)),
            pl.BlockSpec((_R, D), lambda i: (i, 0)),
            pl.BlockSpec((_R, D), lambda i: (i, 0)),
        ],
        out_shape=[
            jax.ShapeDtypeStruct((N, D), jnp.float32),
            jax.ShapeDtypeStruct((N, D), jnp.float32),
            jax.ShapeDtypeStruct((N, D), jnp.float32),
        ],
    )(acc_parts, g, dinv, b_gcn, W_enc, b_enc, W_mu, b_mu, W_std, b_std, eps)

    return (x_new, mu, std)


# speedup vs baseline: 42.4046x; 1.0835x over previous
import functools

import jax
import jax.numpy as jnp
import numpy as np
from jax import lax
from jax.experimental import pallas as pl
from jax.experimental.pallas import tpu as pltpu
from jax.experimental.pallas import tpu_sc as plsc

N = 10000
E = 320000
D = 128
DEPTH = 5

NC = 2
NS = 16
NW = NC * NS
E_PER = E // NW
DEG_CH = 2000
EDGE_CH = 80
N_CHUNKS = E_PER // EDGE_CH
SP_CH = 96
NSETS = 4
SP_ITERS = E_PER // (NSETS * SP_CH)
SP_FULL = SP_ITERS * NSETS
SP_REM = E_PER - SP_FULL * SP_CH
ROWS_PER_TILE = N // NS

@functools.cache
def _make_deg():
    mesh = plsc.VectorSubcoreMesh(core_axis_name="c", subcore_axis_name="s",
                                  num_cores=NC, num_subcores=NS)
    return pl.kernel(
        _deg_body,
        out_type=jax.ShapeDtypeStruct((NC, N), jnp.float32),
        mesh=mesh,
        scratch_types=[
            pltpu.VMEM((E_PER,), jnp.int32),
            pltpu.VMEM((EDGE_CH,), jnp.float32),
            pltpu.VMEM_SHARED((N,), jnp.float32),
            pltpu.SemaphoreType.DMA,
        ],
        compiler_params=pltpu.CompilerParams(use_tc_tiling_on_sc=False),
    )


def _deg_body(ei_hbm, zeros1_hbm, out_hbm, didx_v, ones_v, deg_s, sem):
    cid = lax.axis_index("c")
    sid = lax.axis_index("s")
    wid = sid * NC + cid
    base = wid * E_PER

    def fill(i, _):
        ones_v[pl.ds(i * 16, 16)] = jnp.full((16,), 1.0, jnp.float32)
        return 0

    lax.fori_loop(0, EDGE_CH // 16, fill, 0)

    @pl.when(sid < 10)
    def _():
        pltpu.sync_copy(zeros1_hbm.at[pl.ds(sid * 1000, 1000)],
                        deg_s.at[pl.ds(sid * 1000, 1000)])

    pltpu.sync_copy(ei_hbm.at[1, pl.ds(base, E_PER)], didx_v)
    plsc.subcore_barrier()

    def chunk(c, _):
        pltpu.async_copy(
            ones_v, deg_s.at[didx_v.at[pl.ds(c * EDGE_CH, EDGE_CH)]], sem,
            add=True)
        return 0

    lax.fori_loop(0, N_CHUNKS, chunk, 0)
    pltpu.make_async_copy(ei_hbm.at[1, pl.ds(base, E_PER)], didx_v, sem).wait()
    plsc.subcore_barrier()

    @pl.when(sid < 10)
    def _():
        pltpu.sync_copy(deg_s.at[pl.ds(sid * 1000, 1000)],
                        out_hbm.at[cid, pl.ds(sid * 1000, 1000)])


@functools.cache
def _make_spmm():
    mesh = plsc.VectorSubcoreMesh(core_axis_name="c", subcore_axis_name="s",
                                  num_cores=NC, num_subcores=NS)
    return pl.kernel(
        _spmm_body,
        out_type=jax.ShapeDtypeStruct((NC, N, D), jnp.float32),
        mesh=mesh,
        scratch_types=(
            [pltpu.VMEM((SP_CH,), jnp.int32) for _ in range(NSETS)]
            + [pltpu.VMEM((SP_CH,), jnp.int32) for _ in range(NSETS)]
            + [pltpu.VMEM((SP_CH, D), jnp.float32) for _ in range(NSETS)]
            + [pltpu.VMEM_SHARED((N, D), jnp.float32)]
            + [pltpu.SemaphoreType.DMA for _ in range(2 * NSETS)]
        ),
        compiler_params=pltpu.CompilerParams(use_tc_tiling_on_sc=False),
    )


def _spmm_body(ei_hbm, g_hbm, zeros_hbm, out_hbm, *refs):
    sidx = refs[0:NSETS]
    didx = refs[NSETS:2 * NSETS]
    rows = refs[2 * NSETS:3 * NSETS]
    acc_s = refs[3 * NSETS]
    semg = refs[3 * NSETS + 1:3 * NSETS + 1 + NSETS]
    sems = refs[3 * NSETS + 1 + NSETS:3 * NSETS + 1 + 2 * NSETS]

    cid = lax.axis_index("c")
    sid = lax.axis_index("s")
    wid = sid * NC + cid
    base = wid * E_PER
    row0 = sid * ROWS_PER_TILE

    pltpu.sync_copy(zeros_hbm.at[pl.ds(row0, ROWS_PER_TILE)],
                    acc_s.at[pl.ds(row0, ROWS_PER_TILE)])
    plsc.subcore_barrier()

    def fire(s, e0):
        pltpu.sync_copy(ei_hbm.at[0, pl.ds(base + e0, SP_CH)], sidx[s])
        pltpu.sync_copy(ei_hbm.at[1, pl.ds(base + e0, SP_CH)], didx[s])
        pltpu.async_copy(g_hbm.at[sidx[s]], rows[s], semg[s])

    def wait_gather(s):
        pltpu.make_async_copy(zeros_hbm.at[pl.ds(0, SP_CH)], rows[s],
                              semg[s]).wait()

    def fire_scatter(s):
        pltpu.async_copy(rows[s], acc_s.at[didx[s]], sems[s], add=True)

    def drain_scatter(s):
        pltpu.make_async_copy(zeros_hbm.at[pl.ds(0, SP_CH)], rows[s],
                              sems[s]).wait()

    def ring(i, _):
        c0 = i * NSETS
        for s in range(NSETS):
            c = c0 + s

            @pl.when(i >= 1)
            def _():
                drain_scatter(s)

            fire(s, c * SP_CH)
            sw = (s - 2) % NSETS

            @pl.when(c0 + s >= 2)
            def _():
                wait_gather(sw)
                fire_scatter(sw)

        return 0

    lax.fori_loop(0, SP_ITERS, ring, 0)

    for c in (SP_FULL - 2, SP_FULL - 1):
        s = c % NSETS
        wait_gather(s)
        fire_scatter(s)
    if SP_REM:
        eL = base + SP_FULL * SP_CH
        sL = 0
        drain_scatter(sL)
        pltpu.sync_copy(ei_hbm.at[0, pl.ds(eL, SP_REM)],
                        sidx[sL].at[pl.ds(0, SP_REM)])
        pltpu.sync_copy(ei_hbm.at[1, pl.ds(eL, SP_REM)],
                        didx[sL].at[pl.ds(0, SP_REM)])
        pltpu.async_copy(g_hbm.at[sidx[sL].at[pl.ds(0, SP_REM)]],
                         rows[sL].at[pl.ds(0, SP_REM)], semg[sL]).wait()
        pltpu.sync_copy(rows[sL].at[pl.ds(0, SP_REM)],
                        acc_s.at[didx[sL].at[pl.ds(0, SP_REM)]], add=True)
        for s in range(1, NSETS):
            drain_scatter(s)
    else:
        for s in range(NSETS):
            drain_scatter(s)
    plsc.subcore_barrier()
    pltpu.sync_copy(acc_s.at[pl.ds(row0, ROWS_PER_TILE)],
                    out_hbm.at[cid, pl.ds(row0, ROWS_PER_TILE)])


def _scale_body(x_ref, w_ref, degp_ref, g_ref, dinv_ref):
    h0 = jnp.dot(x_ref[...], w_ref[...], preferred_element_type=jnp.float32)
    deg = jnp.sum(degp_ref[...], axis=0) + 1.0
    dinv = lax.rsqrt(deg)
    g_ref[...] = h0 * dinv[:, None]
    dinv_ref[...] = dinv


def _mlp_body(accp_ref, g_ref, dinv_ref, bgcn_ref, wenc_ref, benc_ref,
              wmu_ref, bmu_ref, wstd_ref, bstd_ref, eps_ref,
              xn_ref, mu_ref, std_ref):
    acc = accp_ref[0] + accp_ref[1]
    h = dinv_ref[...][:, None] * (acc + g_ref[...]) + bgcn_ref[...][None, :]
    for i in range(DEPTH):
        z = jnp.dot(h, wenc_ref[i], preferred_element_type=jnp.float32)
        h = jax.nn.sigmoid(z + benc_ref[i][None, :])
    mu = jnp.dot(h, wmu_ref[...], preferred_element_type=jnp.float32)
    mu = mu + bmu_ref[...][None, :]
    s = jnp.dot(h, wstd_ref[...], preferred_element_type=jnp.float32)
    s = s + bstd_ref[...][None, :] - 5.0
    std = jnp.maximum(s, 0.0) + jnp.log1p(jnp.exp(-jnp.abs(s)))
    mu_ref[...] = mu
    std_ref[...] = std
    xn_ref[...] = mu + std * eps_ref[...]


_R = 1024
_G = (N + _R - 1) // _R


@functools.cache
def _eps_np():
    with jax.ensure_compile_time_eval():
        with jax.default_device(jax.devices("cpu")[0]):
            return np.asarray(jax.random.uniform(
                jax.random.key(42), (N, D), dtype=jnp.float32))


_ZEROS_ND = np.zeros((N, D), np.float32)
_ZEROS_N = np.zeros((N,), np.float32)


def kernel(x, edge_index, W_gcn, b_gcn, W_enc, b_enc, W_mu, b_mu, W_std,
           b_std):
    deg_parts = _make_deg()(edge_index, _ZEROS_N)

    g, dinv = pl.pallas_call(
        _scale_body,
        grid=(_G,),
        in_specs=[
            pl.BlockSpec((_R, D), lambda i: (i, 0)),
            pl.BlockSpec((D, D), lambda i: (0, 0)),
            pl.BlockSpec((NC, _R), lambda i: (0, i)),
        ],
        out_specs=[
            pl.BlockSpec((_R, D), lambda i: (i, 0)),
            pl.BlockSpec((_R,), lambda i: (i,)),
        ],
        out_shape=[
            jax.ShapeDtypeStruct((N, D), jnp.float32),
            jax.ShapeDtypeStruct((N,), jnp.float32),
        ],
    )(x, W_gcn, deg_parts)

    acc_parts = _make_spmm()(edge_index, g, _ZEROS_ND)

    eps = _eps_np()

    x_new, mu, std = pl.pallas_call(
        _mlp_body,
        grid=(_G,),
        in_specs=[
            pl.BlockSpec((NC, _R, D), lambda i: (0, i, 0)),
            pl.BlockSpec((_R, D), lambda i: (i, 0)),
            pl.BlockSpec((_R,), lambda i: (i,)),
            pl.BlockSpec((D,), lambda i: (0,)),
            pl.BlockSpec((DEPTH, D, D), lambda i: (0, 0, 0)),
            pl.BlockSpec((DEPTH, D), lambda i: (0, 0)),
            pl.BlockSpec((D, D), lambda i: (0, 0)),
            pl.BlockSpec((D,), lambda i: (0,)),
            pl.BlockSpec((D, D), lambda i: (0, 0)),
            pl.BlockSpec((D,), lambda i: (0,)),
            pl.BlockSpec((_R, D), lambda i: (i, 0)),
        ],
        out_specs=[
            pl.BlockSpec((_R, D), lambda i: (i, 0)),
            pl.BlockSpec((_R, D), lambda i: (i, 0)),
            pl.BlockSpec((_R, D), lambda i: (i, 0)),
        ],
        out_shape=[
            jax.ShapeDtypeStruct((N, D), jnp.float32),
            jax.ShapeDtypeStruct((N, D), jnp.float32),
            jax.ShapeDtypeStruct((N, D), jnp.float32),
        ],
    )(acc_parts, g, dinv, b_gcn, W_enc, b_enc, W_mu, b_mu, W_std, b_std, eps)

    return (x_new, mu, std)
